# baseline (speedup 1.0000x reference)
import functools
import math

import numpy as np
import jax
import jax.numpy as jnp
from jax.experimental import pallas as pl

_K = 2
_EPS = 1e-08
_TN = 512
_HI = jax.lax.Precision.HIGHEST


def _dot(a, b, prec=None):
    return jax.lax.dot(a, b, precision=prec, preferred_element_type=jnp.float32)


def _body(n_grid, n_tokens, m_experts, b_code, x_ref, h_ref, vt_ref,
          s_ref, hs_ref, idx_ref, scal_ref):
    g = pl.program_id(0)
    h = h_ref[...]
    energy = _dot(h * h, s_ref[...], _HI)

    iota_m = jax.lax.broadcasted_iota(jnp.int32, (h.shape[0], m_experts), 1)
    riota = (m_experts - 1) - iota_m
    e0 = jnp.max(energy, axis=1, keepdims=True)
    m0 = (m_experts - 1) - jnp.max(
        jnp.where(energy == e0, riota, -1), axis=1, keepdims=True)
    masked = jnp.where(iota_m == m0, -jnp.inf, energy)
    e1 = jnp.max(masked, axis=1, keepdims=True)
    m1 = (m_experts - 1) - jnp.max(
        jnp.where(masked == e1, riota, -1), axis=1, keepdims=True)

    mb = m_experts * b_code
    jexp = jax.lax.broadcasted_iota(jnp.int32, (h.shape[0], mb), 1) // b_code
    a0 = jnp.where(jexp == m0, h, 0.0)
    a1 = jnp.where(jexp == m1, h, 0.0)
    h_masked = (a0 + a1).astype(jnp.bfloat16)
    x_hat = jax.lax.dot_general(
        h_masked, vt_ref[...], (((1,), (1,)), ((), ())),
        preferred_element_type=jnp.float32)
    x = x_ref[...]
    resid = x - x_hat

    def _fold(t):
        w = t.shape[1]
        while w > b_code:
            w //= 2
            t = t[:, :w] + t[:, w:]
        return t

    hs_ref[...] = jnp.concatenate(
        [_fold(a0)[:, None, :], _fold(a1)[:, None, :]], axis=1)

    iota_k = jax.lax.broadcasted_iota(jnp.int32, (h.shape[0], _K), 1)
    idx_ref[...] = jnp.where(iota_k == 0, m0, m1)

    cap_s = jnp.sum(jnp.where((iota_m == m0) | (iota_m == m1), energy, 0.0))
    rec_s = jnp.sum(x_hat * x_hat)
    unc_s = jnp.sum(resid * resid)
    esum = jnp.sum(energy, axis=0, keepdims=True)
    il = jax.lax.broadcasted_iota(jnp.int32, (1, 128), 1)
    stepvec = ((il == 0).astype(jnp.float32) * cap_s
               + (il == 1).astype(jnp.float32) * rec_s
               + (il == 2).astype(jnp.float32) * unc_s
               + jnp.concatenate(
                   [jnp.zeros((1, 8), jnp.float32), esum,
                    jnp.zeros((1, 128 - 8 - m_experts), jnp.float32)],
                   axis=1))

    @pl.when(g == 0)
    def _():
        scal_ref[...] = stepvec

    @pl.when(g > 0)
    def _():
        scal_ref[...] = scal_ref[...] + stepvec

    @pl.when(g == n_grid - 1)
    def _():
        acc = scal_ref[...]
        n_f = float(n_tokens)
        emask = ((il >= 8) & (il < 8 + m_experts)).astype(jnp.float32)
        avg = acc * emask / n_f
        denom = jnp.maximum(jnp.sum(avg), _EPS)
        probs = jnp.maximum(avg / denom, _EPS)
        ent = -jnp.sum(emask * probs * jnp.log(probs)) / math.log(m_experts)
        cap = jnp.sum(acc * (il == 0).astype(jnp.float32)) / n_f
        rec = jnp.sum(acc * (il == 1).astype(jnp.float32)) / n_f
        unc = jnp.sum(acc * (il == 2).astype(jnp.float32)) / n_f
        aux = unc + 0.5 * (1.0 - ent)
        scal_ref[...] = ((il == 0).astype(jnp.float32) * cap
                         + (il == 1).astype(jnp.float32) * rec
                         + (il == 2).astype(jnp.float32) * unc
                         + (il == 3).astype(jnp.float32) * ent
                         + (il == 4).astype(jnp.float32) * aux)


@functools.partial(jax.jit, static_argnames=("interpret",))
def kernel(x_flat, h_all, V, interpret=False):
    n, d = x_flat.shape
    _, m, b = h_all.shape
    mb = m * b
    h2 = h_all.reshape(n, mb)
    vt = V.reshape(d, mb).astype(jnp.bfloat16)
    tn = min(_TN, n)
    n_grid = n // tn

    j = np.arange(mb)
    s_np = (j[:, None] // b == np.arange(m)[None, :]).astype(np.float32)

    body = functools.partial(_body, n_grid, n, m, b)
    hs, idx, scal = pl.pallas_call(
        body,
        grid=(n_grid,),
        in_specs=[
            pl.BlockSpec((tn, d), lambda g: (g, 0)),
            pl.BlockSpec((tn, mb), lambda g: (g, 0)),
            pl.BlockSpec((d, mb), lambda g: (0, 0)),
            pl.BlockSpec((mb, m), lambda g: (0, 0)),
        ],
        out_specs=[
            pl.BlockSpec((tn, _K, b), lambda g: (g, 0, 0)),
            pl.BlockSpec((tn, _K), lambda g: (g, 0)),
            pl.BlockSpec((1, 128), lambda g: (0, 0)),
        ],
        out_shape=[
            jax.ShapeDtypeStruct((n, _K, b), jnp.float32),
            jax.ShapeDtypeStruct((n, _K), jnp.int32),
            jax.ShapeDtypeStruct((1, 128), jnp.float32),
        ],
        interpret=interpret,
    )(x_flat, h2, vt, s_np)

    return (hs, idx, scal[0, 0], scal[0, 1], scal[0, 2],
            scal[0, 3], scal[0, 4])


# speedup vs baseline: 1.1279x; 1.1279x over previous
import functools
import math

import numpy as np
import jax
import jax.numpy as jnp
from jax.experimental import pallas as pl

_K = 2
_EPS = 1e-08
_TN = 512
_HI = jax.lax.Precision.HIGHEST


def _dot(a, b, prec=None):
    return jax.lax.dot(a, b, precision=prec, preferred_element_type=jnp.float32)


def _body(n_grid, n_tokens, m_experts, b_code, x_ref, h_ref, vt_ref,
          s_ref, hs_ref, idx_ref, scal_ref):
    g = pl.program_id(0)
    h = h_ref[...]
    hh = h * h
    s_b = s_ref[...]
    hi = hh.astype(jnp.bfloat16)
    r1 = hh - hi.astype(jnp.float32)
    mid = r1.astype(jnp.bfloat16)
    lo = (r1 - mid.astype(jnp.float32)).astype(jnp.bfloat16)
    energy = (_dot(hi, s_b) + _dot(mid, s_b)) + _dot(lo, s_b)

    iota_m = jax.lax.broadcasted_iota(jnp.int32, (h.shape[0], m_experts), 1)
    riota = (m_experts - 1) - iota_m
    e0 = jnp.max(energy, axis=1, keepdims=True)
    m0 = (m_experts - 1) - jnp.max(
        jnp.where(energy == e0, riota, -1), axis=1, keepdims=True)
    masked = jnp.where(iota_m == m0, -jnp.inf, energy)
    e1 = jnp.max(masked, axis=1, keepdims=True)
    m1 = (m_experts - 1) - jnp.max(
        jnp.where(masked == e1, riota, -1), axis=1, keepdims=True)

    mb = m_experts * b_code
    jexp = jax.lax.broadcasted_iota(jnp.int32, (h.shape[0], mb), 1) // b_code
    a0 = jnp.where(jexp == m0, h, 0.0)
    a1 = jnp.where(jexp == m1, h, 0.0)
    h_masked = (a0 + a1).astype(jnp.bfloat16)
    x_hat = jax.lax.dot_general(
        h_masked, vt_ref[...], (((1,), (1,)), ((), ())),
        preferred_element_type=jnp.float32)
    x = x_ref[...]
    resid = x - x_hat

    def _fold(t):
        w = t.shape[1]
        while w > b_code:
            w //= 2
            t = t[:, :w] + t[:, w:]
        return t

    hs_ref[...] = jnp.concatenate([_fold(a0), _fold(a1)], axis=1)

    iota_k = jax.lax.broadcasted_iota(jnp.int32, (h.shape[0], _K), 1)
    idx_ref[...] = jnp.where(iota_k == 0, m0, m1)

    cap_s = jnp.sum(jnp.where((iota_m == m0) | (iota_m == m1), energy, 0.0))
    rec_s = jnp.sum(x_hat * x_hat)
    unc_s = jnp.sum(resid * resid)
    esum = jnp.sum(energy, axis=0, keepdims=True)
    il = jax.lax.broadcasted_iota(jnp.int32, (1, 128), 1)
    stepvec = ((il == 0).astype(jnp.float32) * cap_s
               + (il == 1).astype(jnp.float32) * rec_s
               + (il == 2).astype(jnp.float32) * unc_s
               + jnp.concatenate(
                   [jnp.zeros((1, 8), jnp.float32), esum,
                    jnp.zeros((1, 128 - 8 - m_experts), jnp.float32)],
                   axis=1))

    @pl.when(g == 0)
    def _():
        scal_ref[...] = stepvec

    @pl.when(g > 0)
    def _():
        scal_ref[...] = scal_ref[...] + stepvec

    @pl.when(g == n_grid - 1)
    def _():
        acc = scal_ref[...]
        n_f = float(n_tokens)
        emask = ((il >= 8) & (il < 8 + m_experts)).astype(jnp.float32)
        avg = acc * emask / n_f
        denom = jnp.maximum(jnp.sum(avg), _EPS)
        probs = jnp.maximum(avg / denom, _EPS)
        ent = -jnp.sum(emask * probs * jnp.log(probs)) / math.log(m_experts)
        cap = jnp.sum(acc * (il == 0).astype(jnp.float32)) / n_f
        rec = jnp.sum(acc * (il == 1).astype(jnp.float32)) / n_f
        unc = jnp.sum(acc * (il == 2).astype(jnp.float32)) / n_f
        aux = unc + 0.5 * (1.0 - ent)
        scal_ref[...] = ((il == 0).astype(jnp.float32) * cap
                         + (il == 1).astype(jnp.float32) * rec
                         + (il == 2).astype(jnp.float32) * unc
                         + (il == 3).astype(jnp.float32) * ent
                         + (il == 4).astype(jnp.float32) * aux)


@functools.partial(jax.jit, static_argnames=("interpret",))
def kernel(x_flat, h_all, V, interpret=False):
    n, d = x_flat.shape
    _, m, b = h_all.shape
    mb = m * b
    h2 = h_all.reshape(n, mb)
    vt = V.reshape(d, mb).astype(jnp.bfloat16)
    tn = min(_TN, n)
    n_grid = n // tn

    j = np.arange(mb)
    s_np = jnp.asarray(
        (j[:, None] // b == np.arange(m)[None, :]).astype(np.float32),
        dtype=jnp.bfloat16)

    body = functools.partial(_body, n_grid, n, m, b)
    hs, idx, scal = pl.pallas_call(
        body,
        grid=(n_grid,),
        in_specs=[
            pl.BlockSpec((tn, d), lambda g: (g, 0)),
            pl.BlockSpec((tn, mb), lambda g: (g, 0)),
            pl.BlockSpec((d, mb), lambda g: (0, 0)),
            pl.BlockSpec((mb, m), lambda g: (0, 0)),
        ],
        out_specs=[
            pl.BlockSpec((tn, _K * b), lambda g: (g, 0)),
            pl.BlockSpec((tn, _K), lambda g: (g, 0)),
            pl.BlockSpec((1, 128), lambda g: (0, 0)),
        ],
        out_shape=[
            jax.ShapeDtypeStruct((n, _K * b), jnp.float32),
            jax.ShapeDtypeStruct((n, _K), jnp.int32),
            jax.ShapeDtypeStruct((1, 128), jnp.float32),
        ],
        interpret=interpret,
    )(x_flat, h2, vt, s_np)

    return (hs.reshape(n, _K, b), idx, scal[0, 0], scal[0, 1], scal[0, 2],
            scal[0, 3], scal[0, 4])


# speedup vs baseline: 1.1519x; 1.0213x over previous
import functools
import math

import numpy as np
import jax
import jax.numpy as jnp
from jax.experimental import pallas as pl

_K = 2
_EPS = 1e-08
_TN = 1024
_HI = jax.lax.Precision.HIGHEST


def _dot(a, b, prec=None):
    return jax.lax.dot(a, b, precision=prec, preferred_element_type=jnp.float32)


def _body(n_grid, n_tokens, m_experts, b_code, x_ref, h_ref, vt_ref,
          s_ref, hs_ref, idx_ref, scal_ref):
    g = pl.program_id(0)
    h = h_ref[...]
    hh = h * h
    s_b = s_ref[...]
    hi = hh.astype(jnp.bfloat16)
    r1 = hh - hi.astype(jnp.float32)
    mid = r1.astype(jnp.bfloat16)
    lo = (r1 - mid.astype(jnp.float32)).astype(jnp.bfloat16)
    energy = (_dot(hi, s_b) + _dot(mid, s_b)) + _dot(lo, s_b)

    iota_m = jax.lax.broadcasted_iota(jnp.int32, (h.shape[0], m_experts), 1)
    riota = (m_experts - 1) - iota_m
    e0 = jnp.max(energy, axis=1, keepdims=True)
    m0 = (m_experts - 1) - jnp.max(
        jnp.where(energy == e0, riota, -1), axis=1, keepdims=True)
    masked = jnp.where(iota_m == m0, -jnp.inf, energy)
    e1 = jnp.max(masked, axis=1, keepdims=True)
    m1 = (m_experts - 1) - jnp.max(
        jnp.where(masked == e1, riota, -1), axis=1, keepdims=True)

    mb = m_experts * b_code
    jexp = jax.lax.broadcasted_iota(jnp.int32, (h.shape[0], mb), 1) // b_code
    a0 = jnp.where(jexp == m0, h, 0.0)
    a1 = jnp.where(jexp == m1, h, 0.0)
    h_masked = (a0 + a1).astype(jnp.bfloat16)
    x_hat = jax.lax.dot_general(
        h_masked, vt_ref[...], (((1,), (1,)), ((), ())),
        preferred_element_type=jnp.float32)
    x = x_ref[...]
    resid = x - x_hat

    def _fold(t):
        w = t.shape[1]
        while w > b_code:
            w //= 2
            t = t[:, :w] + t[:, w:]
        return t

    hs_ref[...] = jnp.concatenate([_fold(a0), _fold(a1)], axis=1)

    iota_k = jax.lax.broadcasted_iota(jnp.int32, (h.shape[0], _K), 1)
    idx_ref[...] = jnp.where(iota_k == 0, m0, m1)

    cap_s = jnp.sum(jnp.where((iota_m == m0) | (iota_m == m1), energy, 0.0))
    rec_s = jnp.sum(x_hat * x_hat)
    unc_s = jnp.sum(resid * resid)
    esum = jnp.sum(energy, axis=0, keepdims=True)
    il = jax.lax.broadcasted_iota(jnp.int32, (1, 128), 1)
    stepvec = ((il == 0).astype(jnp.float32) * cap_s
               + (il == 1).astype(jnp.float32) * rec_s
               + (il == 2).astype(jnp.float32) * unc_s
               + jnp.concatenate(
                   [jnp.zeros((1, 8), jnp.float32), esum,
                    jnp.zeros((1, 128 - 8 - m_experts), jnp.float32)],
                   axis=1))

    @pl.when(g == 0)
    def _():
        scal_ref[...] = stepvec

    @pl.when(g > 0)
    def _():
        scal_ref[...] = scal_ref[...] + stepvec

    @pl.when(g == n_grid - 1)
    def _():
        acc = scal_ref[...]
        n_f = float(n_tokens)
        emask = ((il >= 8) & (il < 8 + m_experts)).astype(jnp.float32)
        avg = acc * emask / n_f
        denom = jnp.maximum(jnp.sum(avg), _EPS)
        probs = jnp.maximum(avg / denom, _EPS)
        ent = -jnp.sum(emask * probs * jnp.log(probs)) / math.log(m_experts)
        cap = jnp.sum(acc * (il == 0).astype(jnp.float32)) / n_f
        rec = jnp.sum(acc * (il == 1).astype(jnp.float32)) / n_f
        unc = jnp.sum(acc * (il == 2).astype(jnp.float32)) / n_f
        aux = unc + 0.5 * (1.0 - ent)
        scal_ref[...] = ((il == 0).astype(jnp.float32) * cap
                         + (il == 1).astype(jnp.float32) * rec
                         + (il == 2).astype(jnp.float32) * unc
                         + (il == 3).astype(jnp.float32) * ent
                         + (il == 4).astype(jnp.float32) * aux)


@functools.partial(jax.jit, static_argnames=("interpret",))
def kernel(x_flat, h_all, V, interpret=False):
    n, d = x_flat.shape
    _, m, b = h_all.shape
    mb = m * b
    h2 = h_all.reshape(n, mb)
    vt = V.reshape(d, mb).astype(jnp.bfloat16)
    tn = min(_TN, n)
    n_grid = n // tn

    j = np.arange(mb)
    s_np = jnp.asarray(
        (j[:, None] // b == np.arange(m)[None, :]).astype(np.float32),
        dtype=jnp.bfloat16)

    body = functools.partial(_body, n_grid, n, m, b)
    hs, idx, scal = pl.pallas_call(
        body,
        grid=(n_grid,),
        in_specs=[
            pl.BlockSpec((tn, d), lambda g: (g, 0)),
            pl.BlockSpec((tn, mb), lambda g: (g, 0)),
            pl.BlockSpec((d, mb), lambda g: (0, 0)),
            pl.BlockSpec((mb, m), lambda g: (0, 0)),
        ],
        out_specs=[
            pl.BlockSpec((tn, _K * b), lambda g: (g, 0)),
            pl.BlockSpec((tn, _K), lambda g: (g, 0)),
            pl.BlockSpec((1, 128), lambda g: (0, 0)),
        ],
        out_shape=[
            jax.ShapeDtypeStruct((n, _K * b), jnp.float32),
            jax.ShapeDtypeStruct((n, _K), jnp.int32),
            jax.ShapeDtypeStruct((1, 128), jnp.float32),
        ],
        interpret=interpret,
    )(x_flat, h2, vt, s_np)

    return (hs.reshape(n, _K, b), idx, scal[0, 0], scal[0, 1], scal[0, 2],
            scal[0, 3], scal[0, 4])
